# baseline (speedup 1.0000x reference)
import functools

import jax
import jax.numpy as jnp
from jax import lax
from jax.experimental import pallas as pl
from jax.experimental.pallas import tpu as pltpu
from jax.experimental.pallas import tpu_sc as plsc

NUM_GENES = 2000
EMBED_DIM = 32
BATCH = 1024
NC = 2
NS = 16
NW = NC * NS
GROUP = 8
NGROUPS = NUM_GENES // GROUP
GROUPS_PER_W = -(-NGROUPS // NW)


def _norm_body(x_ref, o_ref):
    x = x_ref[...]
    for k in range(4):
        c = x[:, k * EMBED_DIM:(k + 1) * EMBED_DIM]
        s = jnp.sum(c * c, axis=1, keepdims=True)
        denom = jnp.maximum(jnp.sqrt(s), 1e-12)
        o_ref[:, k * EMBED_DIM:(k + 1) * EMBED_DIM] = c / denom


def _normalize_table(embedding_mat):
    emb2d = embedding_mat.reshape(NUM_GENES, 4 * EMBED_DIM)
    return pl.pallas_call(
        _norm_body,
        out_shape=jax.ShapeDtypeStruct((NUM_GENES, 4 * EMBED_DIM), jnp.float32),
    )(emb2d)


_sc_mesh = plsc.VectorSubcoreMesh(
    core_axis_name="c", subcore_axis_name="s", num_cores=NC, num_subcores=NS
)


@functools.partial(
    pl.kernel,
    out_type=jax.ShapeDtypeStruct((NUM_GENES, EMBED_DIM, BATCH), jnp.float32),
    mesh=_sc_mesh,
    scratch_types=[
        pltpu.VMEM((GROUP * BATCH,), jnp.int32),
        pltpu.VMEM((GROUP * 4 * EMBED_DIM,), jnp.float32),
        pltpu.VMEM((EMBED_DIM, BATCH), jnp.float32),
        pltpu.VMEM_SHARED((NS, 2, EMBED_DIM, BATCH), jnp.float32),
        pltpu.SemaphoreType.DMA,
        pltpu.SemaphoreType.DMA,
    ],
    compiler_params=pltpu.CompilerParams(
        use_tc_tiling_on_sc=True, needs_layout_passes=False
    ),
)
def _sc_select(table_hbm, gst_hbm, out_hbm, kv, slab, ob, sh, ds0, ds1):
    wid = lax.axis_index("s") * NC + lax.axis_index("c")
    sid = lax.axis_index("s")
    dsem = (ds0, ds1)

    def _wait_write(slot):
        pltpu.make_async_copy(
            sh.at[sid, slot], out_hbm.at[0], dsem[slot]
        ).wait()

    def group_body(t, carry):
        g = t * NW + wid

        @pl.when(g < NGROUPS)
        def _():
            m0 = g * GROUP
            pltpu.sync_copy(gst_hbm.at[pl.ds(m0 * BATCH, GROUP * BATCH)], kv)
            pltpu.sync_copy(
                table_hbm.at[pl.ds(m0 * 4 * EMBED_DIM, GROUP * 4 * EMBED_DIM)],
                slab,
            )
            for r in range(GROUP):
                slot = r % 2

                if r >= 2:
                    _wait_write(slot)
                else:
                    @pl.when(t > 0)
                    def _(slot=slot):
                        _wait_write(slot)

                @plsc.parallel_loop(0, BATCH // 16, unroll=4)
                def vec_body(j, r=r):
                    kvec = kv[pl.ds(r * BATCH + j * 16, 16)]
                    base = kvec * EMBED_DIM + (r * 4 * EMBED_DIM)
                    for d in range(EMBED_DIM):
                        val = plsc.load_gather(slab, [base + d])
                        ob[d, pl.ds(j * 16, 16)] = val

                pltpu.sync_copy(ob, sh.at[sid, slot])
                pltpu.async_copy(sh.at[sid, slot], out_hbm.at[m0 + r],
                                 dsem[slot])
        return carry

    lax.fori_loop(0, GROUPS_PER_W, group_body, 0)
    for slot in range(2):
        _wait_write(slot)


def kernel(gene_seq, embedding_mat):
    table = _normalize_table(embedding_mat).reshape(-1)
    gst = jnp.transpose(gene_seq).reshape(-1)
    out = _sc_select(table, gst)
    return jnp.transpose(out, (2, 0, 1))


# speedup vs baseline: 3.6414x; 3.6414x over previous
import functools

import jax
import jax.numpy as jnp
from jax import lax
from jax.experimental import pallas as pl
from jax.experimental.pallas import tpu as pltpu
from jax.experimental.pallas import tpu_sc as plsc

NUM_GENES = 2000
EMBED_DIM = 32
BATCH = 1024
NC = 2
NS = 16
NW = NC * NS
GROUP = 8
SC_GENES = 256
SC_GROUPS = SC_GENES // GROUP
TC_GENES = NUM_GENES - SC_GENES
TC_BLOCKS = TC_GENES // GROUP


def _norm_body(x_ref, o_ref):
    x = x_ref[...]
    for k in range(4):
        c = x[:, k * EMBED_DIM:(k + 1) * EMBED_DIM]
        s = jnp.sum(c * c, axis=1, keepdims=True)
        denom = jnp.maximum(jnp.sqrt(s), 1e-12)
        o_ref[:, k * EMBED_DIM:(k + 1) * EMBED_DIM] = c / denom


def _normalize_table(embedding_mat):
    emb2d = embedding_mat.reshape(NUM_GENES, 4 * EMBED_DIM)
    return pl.pallas_call(
        _norm_body,
        out_shape=jax.ShapeDtypeStruct((NUM_GENES, 4 * EMBED_DIM), jnp.float32),
    )(emb2d)


_sc_mesh = plsc.VectorSubcoreMesh(
    core_axis_name="c", subcore_axis_name="s", num_cores=NC, num_subcores=NS
)


@functools.partial(
    pl.kernel,
    out_type=jax.ShapeDtypeStruct((SC_GENES, EMBED_DIM, BATCH), jnp.float32),
    mesh=_sc_mesh,
    scratch_types=[
        pltpu.VMEM((GROUP * BATCH,), jnp.int32),
        pltpu.VMEM((GROUP * 4 * EMBED_DIM,), jnp.float32),
        pltpu.VMEM((EMBED_DIM, BATCH), jnp.float32),
        pltpu.VMEM((EMBED_DIM, BATCH), jnp.float32),
        pltpu.SemaphoreType.DMA,
        pltpu.SemaphoreType.DMA,
    ],
    compiler_params=pltpu.CompilerParams(
        use_tc_tiling_on_sc=True, needs_layout_passes=False
    ),
)
def _sc_select(table_hbm, gst_hbm, out_hbm, kv, slab, ob0, ob1, ws0, ws1):
    wid = lax.axis_index("s") * NC + lax.axis_index("c")
    ob = (ob0, ob1)
    ws = (ws0, ws1)
    m0 = wid * GROUP
    pltpu.sync_copy(gst_hbm.at[pl.ds(m0 * BATCH, GROUP * BATCH)], kv)
    pltpu.sync_copy(
        table_hbm.at[pl.ds(m0 * 4 * EMBED_DIM, GROUP * 4 * EMBED_DIM)], slab
    )
    for r in range(GROUP):
        buf = ob[r % 2]
        if r >= 2:
            pltpu.make_async_copy(buf, out_hbm.at[0], ws[r % 2]).wait()

        @plsc.parallel_loop(0, BATCH // 16, unroll=4)
        def vec_body(j, r=r, buf=buf):
            kvec = kv[pl.ds(r * BATCH + j * 16, 16)]
            base = kvec * EMBED_DIM + (r * 4 * EMBED_DIM)
            for d in range(EMBED_DIM):
                val = plsc.load_gather(slab, [base + d])
                buf[d, pl.ds(j * 16, 16)] = val

        pltpu.async_copy(buf, out_hbm.at[m0 + r], ws[r % 2])
    for r in range(2):
        pltpu.make_async_copy(ob[r], out_hbm.at[0], ws[r]).wait()


def _tc_select_body(gst_ref, e_ref, o_ref):
    for r in range(GROUP):
        krow = gst_ref[r, :].reshape(1, BATCH)
        onehot = (
            lax.broadcasted_iota(jnp.int32, (4, BATCH), 0) == krow
        ).astype(jnp.float32)
        esub = e_ref[r * 4:(r + 1) * 4, :]
        o_ref[r] = lax.dot_general(
            esub, onehot, (((0,), (0,)), ((), ())),
            preferred_element_type=jnp.float32,
            precision=lax.Precision.HIGHEST,
        )


def _tc_select(gst_tc, e_all):
    return pl.pallas_call(
        _tc_select_body,
        grid=(TC_BLOCKS,),
        in_specs=[
            pl.BlockSpec((GROUP, BATCH), lambda i: (i, 0)),
            pl.BlockSpec((GROUP * 4, EMBED_DIM), lambda i: (i, 0)),
        ],
        out_specs=pl.BlockSpec(
            (GROUP, EMBED_DIM, BATCH), lambda i: (SC_GROUPS + i, 0, 0)
        ),
        out_shape=jax.ShapeDtypeStruct(
            (NUM_GENES, EMBED_DIM, BATCH), jnp.float32
        ),
    )(gst_tc, e_all)


def _paste_body(big_ref, sc_ref, o_ref):
    o_ref[...] = sc_ref[...]


def _paste(big, sc_out):
    return pl.pallas_call(
        _paste_body,
        grid=(SC_GROUPS,),
        in_specs=[
            pl.BlockSpec(memory_space=pl.ANY),
            pl.BlockSpec((GROUP, EMBED_DIM, BATCH), lambda i: (i, 0, 0)),
        ],
        out_specs=pl.BlockSpec(
            (GROUP, EMBED_DIM, BATCH), lambda i: (i, 0, 0)
        ),
        out_shape=jax.ShapeDtypeStruct(
            (NUM_GENES, EMBED_DIM, BATCH), jnp.float32
        ),
        input_output_aliases={0: 0},
    )(big, sc_out)


def kernel(gene_seq, embedding_mat):
    table = _normalize_table(embedding_mat)
    gst = jnp.transpose(gene_seq)
    sc_out = _sc_select(
        table[:SC_GENES].reshape(-1), gst[:SC_GENES].reshape(-1)
    )
    e_all = table[SC_GENES:].reshape(TC_GENES * 4, EMBED_DIM)
    big = _tc_select(gst[SC_GENES:], e_all)
    out = _paste(big, sc_out)
    return jnp.transpose(out, (2, 0, 1))


# speedup vs baseline: 3.9905x; 1.0959x over previous
import functools

import jax
import jax.numpy as jnp
from jax import lax
from jax.experimental import pallas as pl
from jax.experimental.pallas import tpu as pltpu
from jax.experimental.pallas import tpu_sc as plsc

NUM_GENES = 2000
EMBED_DIM = 32
BATCH = 1024
NC = 2
NS = 16
NW = NC * NS
GROUP = 8
SC_GENES = 256
SC_GROUPS = SC_GENES // GROUP
TC_GENES = NUM_GENES - SC_GENES
TC_BLOCKS = TC_GENES // GROUP


def _norm_body(x_ref, o_ref):
    x = x_ref[...]
    for k in range(4):
        c = x[:, k * EMBED_DIM:(k + 1) * EMBED_DIM]
        s = jnp.sum(c * c, axis=1, keepdims=True)
        denom = jnp.maximum(jnp.sqrt(s), 1e-12)
        o_ref[:, k * EMBED_DIM:(k + 1) * EMBED_DIM] = c / denom


def _normalize_table(embedding_mat):
    emb2d = embedding_mat.reshape(NUM_GENES, 4 * EMBED_DIM)
    return pl.pallas_call(
        _norm_body,
        out_shape=jax.ShapeDtypeStruct((NUM_GENES, 4 * EMBED_DIM), jnp.float32),
    )(emb2d)


_sc_mesh = plsc.VectorSubcoreMesh(
    core_axis_name="c", subcore_axis_name="s", num_cores=NC, num_subcores=NS
)


@functools.partial(
    pl.kernel,
    out_type=jax.ShapeDtypeStruct((SC_GENES, EMBED_DIM, BATCH), jnp.float32),
    mesh=_sc_mesh,
    scratch_types=[
        pltpu.VMEM((GROUP * BATCH,), jnp.int32),
        pltpu.VMEM((GROUP * 4 * EMBED_DIM,), jnp.float32),
        pltpu.VMEM((EMBED_DIM, BATCH), jnp.float32),
        pltpu.VMEM((EMBED_DIM, BATCH), jnp.float32),
        pltpu.SemaphoreType.DMA,
        pltpu.SemaphoreType.DMA,
    ],
    compiler_params=pltpu.CompilerParams(
        use_tc_tiling_on_sc=True, needs_layout_passes=False
    ),
)
def _sc_select(table_hbm, gst_hbm, out_hbm, kv, slab, ob0, ob1, ws0, ws1):
    wid = lax.axis_index("s") * NC + lax.axis_index("c")
    ob = (ob0, ob1)
    ws = (ws0, ws1)
    m0 = wid * GROUP
    pltpu.sync_copy(gst_hbm.at[pl.ds(m0 * BATCH, GROUP * BATCH)], kv)
    pltpu.sync_copy(
        table_hbm.at[pl.ds(m0 * 4 * EMBED_DIM, GROUP * 4 * EMBED_DIM)], slab
    )
    for r in range(GROUP):
        buf = ob[r % 2]
        if r >= 2:
            pltpu.make_async_copy(buf, out_hbm.at[0], ws[r % 2]).wait()

        @plsc.parallel_loop(0, BATCH // 16, unroll=4)
        def vec_body(j, r=r, buf=buf):
            kvec = kv[pl.ds(r * BATCH + j * 16, 16)]
            base = kvec * EMBED_DIM + (r * 4 * EMBED_DIM)
            for d in range(EMBED_DIM):
                val = plsc.load_gather(slab, [base + d])
                buf[d, pl.ds(j * 16, 16)] = val

        pltpu.async_copy(buf, out_hbm.at[m0 + r], ws[r % 2])
    for r in range(2):
        pltpu.make_async_copy(ob[r], out_hbm.at[0], ws[r]).wait()


def _tc_select_body(gst_ref, e1_ref, e2_ref, e3_ref, o_ref):
    for r in range(GROUP):
        krow = gst_ref[r, :].reshape(1, BATCH)
        onehot = (
            lax.broadcasted_iota(jnp.int32, (4, BATCH), 0) == krow
        ).astype(jnp.bfloat16)
        acc = None
        for e_ref in (e1_ref, e2_ref, e3_ref):
            part = lax.dot_general(
                e_ref[r * 4:(r + 1) * 4, :], onehot,
                (((0,), (0,)), ((), ())),
                preferred_element_type=jnp.float32,
            )
            acc = part if acc is None else acc + part
        o_ref[r] = acc


def _tc_select(gst_tc, e1, e2, e3):
    espec = pl.BlockSpec((GROUP * 4, EMBED_DIM), lambda i: (i, 0))
    return pl.pallas_call(
        _tc_select_body,
        grid=(TC_BLOCKS,),
        in_specs=[
            pl.BlockSpec((GROUP, BATCH), lambda i: (i, 0)),
            espec, espec, espec,
        ],
        out_specs=pl.BlockSpec(
            (GROUP, EMBED_DIM, BATCH), lambda i: (SC_GROUPS + i, 0, 0)
        ),
        out_shape=jax.ShapeDtypeStruct(
            (NUM_GENES, EMBED_DIM, BATCH), jnp.float32
        ),
    )(gst_tc, e1, e2, e3)


def _paste_body(big_ref, sc_ref, o_ref):
    o_ref[...] = sc_ref[...]


def _paste(big, sc_out):
    return pl.pallas_call(
        _paste_body,
        grid=(SC_GROUPS,),
        in_specs=[
            pl.BlockSpec(memory_space=pl.ANY),
            pl.BlockSpec((GROUP, EMBED_DIM, BATCH), lambda i: (i, 0, 0)),
        ],
        out_specs=pl.BlockSpec(
            (GROUP, EMBED_DIM, BATCH), lambda i: (i, 0, 0)
        ),
        out_shape=jax.ShapeDtypeStruct(
            (NUM_GENES, EMBED_DIM, BATCH), jnp.float32
        ),
        input_output_aliases={0: 0},
    )(big, sc_out)


def kernel(gene_seq, embedding_mat):
    table = _normalize_table(embedding_mat)
    gst = jnp.transpose(gene_seq)
    sc_out = _sc_select(
        table[:SC_GENES].reshape(-1), gst[:SC_GENES].reshape(-1)
    )
    e_all = table[SC_GENES:].reshape(TC_GENES * 4, EMBED_DIM)
    e1 = e_all.astype(jnp.bfloat16)
    r1 = e_all - e1.astype(jnp.float32)
    e2 = r1.astype(jnp.bfloat16)
    e3 = (r1 - e2.astype(jnp.float32)).astype(jnp.bfloat16)
    big = _tc_select(gst[SC_GENES:], e1, e2, e3)
    out = _paste(big, sc_out)
    return jnp.transpose(out, (2, 0, 1))


# speedup vs baseline: 5.0525x; 1.2661x over previous
import functools

import jax
import jax.numpy as jnp
from jax import lax
from jax.experimental import pallas as pl
from jax.experimental.pallas import tpu as pltpu
from jax.experimental.pallas import tpu_sc as plsc

NUM_GENES = 2000
EMBED_DIM = 32
BATCH = 1024
NC = 2
NS = 16
NW = NC * NS
GROUP = 8
SC_GENES = 256
SC_GROUPS = SC_GENES // GROUP
TC_GENES = NUM_GENES - SC_GENES
TC_GROUP = 16
TC_BLOCKS = TC_GENES // TC_GROUP


def _norm_body(x_ref, o_ref):
    x = x_ref[...]
    for k in range(4):
        c = x[:, k * EMBED_DIM:(k + 1) * EMBED_DIM]
        s = jnp.sum(c * c, axis=1, keepdims=True)
        denom = jnp.maximum(jnp.sqrt(s), 1e-12)
        o_ref[:, k * EMBED_DIM:(k + 1) * EMBED_DIM] = c / denom


def _normalize_table(embedding_mat):
    emb2d = embedding_mat.reshape(NUM_GENES, 4 * EMBED_DIM)
    return pl.pallas_call(
        _norm_body,
        out_shape=jax.ShapeDtypeStruct((NUM_GENES, 4 * EMBED_DIM), jnp.float32),
    )(emb2d)


_sc_mesh = plsc.VectorSubcoreMesh(
    core_axis_name="c", subcore_axis_name="s", num_cores=NC, num_subcores=NS
)


@functools.partial(
    pl.kernel,
    out_type=jax.ShapeDtypeStruct((SC_GENES, EMBED_DIM, BATCH), jnp.float32),
    mesh=_sc_mesh,
    scratch_types=[
        pltpu.VMEM((GROUP * BATCH,), jnp.int32),
        pltpu.VMEM((GROUP * 4 * EMBED_DIM,), jnp.float32),
        pltpu.VMEM((EMBED_DIM, BATCH), jnp.float32),
        pltpu.VMEM((EMBED_DIM, BATCH), jnp.float32),
        pltpu.SemaphoreType.DMA,
        pltpu.SemaphoreType.DMA,
    ],
    compiler_params=pltpu.CompilerParams(
        use_tc_tiling_on_sc=True, needs_layout_passes=False
    ),
)
def _sc_select(table_hbm, gst_hbm, out_hbm, kv, slab, ob0, ob1, ws0, ws1):
    wid = lax.axis_index("s") * NC + lax.axis_index("c")
    ob = (ob0, ob1)
    ws = (ws0, ws1)
    m0 = wid * GROUP
    pltpu.sync_copy(gst_hbm.at[pl.ds(m0 * BATCH, GROUP * BATCH)], kv)
    pltpu.sync_copy(
        table_hbm.at[pl.ds(m0 * 4 * EMBED_DIM, GROUP * 4 * EMBED_DIM)], slab
    )
    for r in range(GROUP):
        buf = ob[r % 2]
        if r >= 2:
            pltpu.make_async_copy(buf, out_hbm.at[0], ws[r % 2]).wait()

        @plsc.parallel_loop(0, BATCH // 16, unroll=4)
        def vec_body(j, r=r, buf=buf):
            kvec = kv[pl.ds(r * BATCH + j * 16, 16)]
            base = kvec * EMBED_DIM + (r * 4 * EMBED_DIM)
            for d in range(EMBED_DIM):
                val = plsc.load_gather(slab, [base + d])
                buf[d, pl.ds(j * 16, 16)] = val

        pltpu.async_copy(buf, out_hbm.at[m0 + r], ws[r % 2])
    for r in range(2):
        pltpu.make_async_copy(ob[r], out_hbm.at[0], ws[r]).wait()


def _tc_select_body(gst_ref, e1_ref, e2_ref, e3_ref, o_ref):
    for r in range(TC_GROUP):
        krow = gst_ref[r, :].reshape(1, BATCH)
        onehot = (
            lax.broadcasted_iota(jnp.int32, (4, BATCH), 0) == krow
        ).astype(jnp.bfloat16)
        acc = None
        for e_ref in (e1_ref, e2_ref, e3_ref):
            part = lax.dot_general(
                e_ref[r * 4:(r + 1) * 4, :], onehot,
                (((0,), (0,)), ((), ())),
                preferred_element_type=jnp.float32,
            )
            acc = part if acc is None else acc + part
        o_ref[r] = acc


def _tc_select(gst_tc, e1, e2, e3):
    espec = pl.BlockSpec((TC_GROUP * 4, EMBED_DIM), lambda i: (i, 0))
    return pl.pallas_call(
        _tc_select_body,
        grid=(TC_BLOCKS,),
        in_specs=[
            pl.BlockSpec((TC_GROUP, BATCH), lambda i: (i, 0)),
            espec, espec, espec,
        ],
        out_specs=pl.BlockSpec(
            (TC_GROUP, EMBED_DIM, BATCH),
            lambda i: (SC_GENES // TC_GROUP + i, 0, 0),
        ),
        out_shape=jax.ShapeDtypeStruct(
            (NUM_GENES, EMBED_DIM, BATCH), jnp.float32
        ),
    )(gst_tc, e1, e2, e3)


def _paste_body(big_ref, sc_ref, o_ref):
    o_ref[...] = sc_ref[...]


def _paste(big, sc_out):
    return pl.pallas_call(
        _paste_body,
        grid=(SC_GROUPS,),
        in_specs=[
            pl.BlockSpec(memory_space=pl.ANY),
            pl.BlockSpec((GROUP, EMBED_DIM, BATCH), lambda i: (i, 0, 0)),
        ],
        out_specs=pl.BlockSpec(
            (GROUP, EMBED_DIM, BATCH), lambda i: (i, 0, 0)
        ),
        out_shape=jax.ShapeDtypeStruct(
            (NUM_GENES, EMBED_DIM, BATCH), jnp.float32
        ),
        input_output_aliases={0: 0},
    )(big, sc_out)


def kernel(gene_seq, embedding_mat):
    table = _normalize_table(embedding_mat)
    gst = jnp.transpose(gene_seq)
    sc_out = _sc_select(
        table[:SC_GENES].reshape(-1), gst[:SC_GENES].reshape(-1)
    )
    e_all = table[SC_GENES:].reshape(TC_GENES * 4, EMBED_DIM)
    e1 = e_all.astype(jnp.bfloat16)
    r1 = e_all - e1.astype(jnp.float32)
    e2 = r1.astype(jnp.bfloat16)
    e3 = (r1 - e2.astype(jnp.float32)).astype(jnp.bfloat16)
    big = _tc_select(gst[SC_GENES:], e1, e2, e3)
    out = _paste(big, sc_out)
    return jnp.transpose(out, (2, 0, 1))


# speedup vs baseline: 5.7511x; 1.1383x over previous
import functools

import jax
import jax.numpy as jnp
from jax import lax
from jax.experimental import pallas as pl
from jax.experimental.pallas import tpu as pltpu
from jax.experimental.pallas import tpu_sc as plsc

NUM_GENES = 2000
EMBED_DIM = 32
BATCH = 1024
NC = 2
NS = 16
NW = NC * NS
GROUP = 8
SC_GENES = 256
SC_GROUPS = SC_GENES // GROUP
TC_GENES = NUM_GENES - SC_GENES
TC_GROUP = 32
TC_BLOCKS = TC_GENES // TC_GROUP


def _norm_body(x_ref, o_ref):
    x = x_ref[...]
    for k in range(4):
        c = x[:, k * EMBED_DIM:(k + 1) * EMBED_DIM]
        s = jnp.sum(c * c, axis=1, keepdims=True)
        denom = jnp.maximum(jnp.sqrt(s), 1e-12)
        o_ref[:, k * EMBED_DIM:(k + 1) * EMBED_DIM] = c / denom


def _normalize_table(embedding_mat):
    emb2d = embedding_mat.reshape(NUM_GENES, 4 * EMBED_DIM)
    return pl.pallas_call(
        _norm_body,
        out_shape=jax.ShapeDtypeStruct((NUM_GENES, 4 * EMBED_DIM), jnp.float32),
    )(emb2d)


_sc_mesh = plsc.VectorSubcoreMesh(
    core_axis_name="c", subcore_axis_name="s", num_cores=NC, num_subcores=NS
)


@functools.partial(
    pl.kernel,
    out_type=jax.ShapeDtypeStruct((SC_GENES, EMBED_DIM, BATCH), jnp.float32),
    mesh=_sc_mesh,
    scratch_types=[
        pltpu.VMEM((GROUP * BATCH,), jnp.int32),
        pltpu.VMEM((GROUP * 4 * EMBED_DIM,), jnp.float32),
        pltpu.VMEM((EMBED_DIM, BATCH), jnp.float32),
        pltpu.VMEM((EMBED_DIM, BATCH), jnp.float32),
        pltpu.SemaphoreType.DMA,
        pltpu.SemaphoreType.DMA,
    ],
    compiler_params=pltpu.CompilerParams(
        use_tc_tiling_on_sc=True, needs_layout_passes=False
    ),
)
def _sc_select(table_hbm, gst_hbm, out_hbm, kv, slab, ob0, ob1, ws0, ws1):
    wid = lax.axis_index("s") * NC + lax.axis_index("c")
    ob = (ob0, ob1)
    ws = (ws0, ws1)
    m0 = wid * GROUP
    pltpu.sync_copy(gst_hbm.at[pl.ds(m0 * BATCH, GROUP * BATCH)], kv)
    pltpu.sync_copy(
        table_hbm.at[pl.ds(m0 * 4 * EMBED_DIM, GROUP * 4 * EMBED_DIM)], slab
    )
    for r in range(GROUP):
        buf = ob[r % 2]
        if r >= 2:
            pltpu.make_async_copy(buf, out_hbm.at[0], ws[r % 2]).wait()

        @plsc.parallel_loop(0, BATCH // 16, unroll=4)
        def vec_body(j, r=r, buf=buf):
            kvec = kv[pl.ds(r * BATCH + j * 16, 16)]
            base = kvec * EMBED_DIM + (r * 4 * EMBED_DIM)
            for d in range(EMBED_DIM):
                val = plsc.load_gather(slab, [base + d])
                buf[d, pl.ds(j * 16, 16)] = val

        pltpu.async_copy(buf, out_hbm.at[m0 + r], ws[r % 2])
    for r in range(2):
        pltpu.make_async_copy(ob[r], out_hbm.at[0], ws[r]).wait()


def _tc_select_body(gst_ref, e1_ref, e2_ref, e3_ref, o_ref):
    for r in range(TC_GROUP):
        krow = gst_ref[r, :].reshape(1, BATCH)
        onehot = (
            lax.broadcasted_iota(jnp.int32, (4, BATCH), 0) == krow
        ).astype(jnp.bfloat16)
        acc = None
        for e_ref in (e1_ref, e2_ref, e3_ref):
            part = lax.dot_general(
                e_ref[r * 4:(r + 1) * 4, :], onehot,
                (((0,), (0,)), ((), ())),
                preferred_element_type=jnp.float32,
            )
            acc = part if acc is None else acc + part
        o_ref[r] = acc


def _tc_select(gst_tc, e1, e2, e3):
    espec = pl.BlockSpec((TC_GROUP * 4, EMBED_DIM), lambda i: (i, 0))
    return pl.pallas_call(
        _tc_select_body,
        grid=(TC_BLOCKS,),
        in_specs=[
            pl.BlockSpec((TC_GROUP, BATCH), lambda i: (i, 0)),
            espec, espec, espec,
        ],
        out_specs=pl.BlockSpec(
            (TC_GROUP, EMBED_DIM, BATCH),
            lambda i: (SC_GENES // TC_GROUP + i, 0, 0),
        ),
        out_shape=jax.ShapeDtypeStruct(
            (NUM_GENES, EMBED_DIM, BATCH), jnp.float32
        ),
    )(gst_tc, e1, e2, e3)


def _paste_body(big_ref, sc_ref, o_ref):
    o_ref[...] = sc_ref[...]


def _paste(big, sc_out):
    return pl.pallas_call(
        _paste_body,
        grid=(SC_GROUPS,),
        in_specs=[
            pl.BlockSpec(memory_space=pl.ANY),
            pl.BlockSpec((GROUP, EMBED_DIM, BATCH), lambda i: (i, 0, 0)),
        ],
        out_specs=pl.BlockSpec(
            (GROUP, EMBED_DIM, BATCH), lambda i: (i, 0, 0)
        ),
        out_shape=jax.ShapeDtypeStruct(
            (NUM_GENES, EMBED_DIM, BATCH), jnp.float32
        ),
        input_output_aliases={0: 0},
    )(big, sc_out)


def kernel(gene_seq, embedding_mat):
    table = _normalize_table(embedding_mat)
    gst = jnp.transpose(gene_seq)
    sc_out = _sc_select(
        table[:SC_GENES].reshape(-1), gst[:SC_GENES].reshape(-1)
    )
    e_all = table[SC_GENES:].reshape(TC_GENES * 4, EMBED_DIM)
    e1 = e_all.astype(jnp.bfloat16)
    r1 = e_all - e1.astype(jnp.float32)
    e2 = r1.astype(jnp.bfloat16)
    e3 = (r1 - e2.astype(jnp.float32)).astype(jnp.bfloat16)
    big = _tc_select(gst[SC_GENES:], e1, e2, e3)
    out = _paste(big, sc_out)
    return jnp.transpose(out, (2, 0, 1))
